# baseline (speedup 1.0000x reference)
import functools

import jax
import jax.numpy as jnp
from jax import lax
from jax.experimental import pallas as pl
from jax.experimental.pallas import tpu as pltpu
from jax.experimental.pallas import tpu_sc as plsc

B, N, C, K, NBLK = 2, 4096, 128, 16, 2
BN_ROWS = B * N
BNK = B * N * K
EPS = 1e-5
TRL = 512
TRB = 512
TRK = 1024

_G2 = BN_ROWS // TRL
_G3 = BN_ROWS // TRB

_NC, _NS = 2, 16
_NW = _NC * _NS
_PER_W = BNK // _NW
_CH = 128
_NCH = _PER_W // _CH


def _mstats(s1_ref, s2_ref, n):
    inv = 1.0 / n
    m = s1_ref[...] * inv
    v = s2_ref[...] * inv - m * m
    return m, lax.rsqrt(v + EPS)


def _acc_init(i, s1_ref, s2_ref):
    @pl.when(i == 0)
    def _():
        s1_ref[...] = jnp.zeros_like(s1_ref)
        s2_ref[...] = jnp.zeros_like(s2_ref)


def _acc(s1_ref, s2_ref, x2):
    s1_ref[...] += jnp.sum(x2, axis=0, keepdims=True)
    s2_ref[...] += jnp.sum(x2 * x2, axis=0, keepdims=True)


def _knn_body(cp_ref, ct_ref, idx_ref):
    b = pl.program_id(0)
    cr = cp_ref[0]
    ct = ct_ref[0]
    dot = jnp.dot(cr, ct, preferred_element_type=jnp.float32)
    sq_all = jnp.sum(ct * ct, axis=0, keepdims=True)
    sq_r = jnp.sum(cr * cr, axis=1, keepdims=True)
    d2 = sq_r + sq_all - 2.0 * dot
    jcol = lax.broadcasted_iota(jnp.int32, (TRK, N), 1)
    cols = []
    for _ in range(K):
        m = jnp.min(d2, axis=1, keepdims=True)
        cand = jnp.where(d2 == m, jcol, N)
        ik = jnp.min(cand, axis=1, keepdims=True)
        cols.append(ik)
        d2 = jnp.where(jcol == ik, jnp.inf, d2)
    idx_ref[0] = jnp.concatenate(cols, axis=1) + b * N


def _knn(cp, ct):
    return pl.pallas_call(
        _knn_body,
        grid=(B, N // TRK),
        in_specs=[
            pl.BlockSpec((1, TRK, 8), lambda b, i: (b, i, 0)),
            pl.BlockSpec((1, 8, N), lambda b, i: (b, 0, 0)),
        ],
        out_specs=pl.BlockSpec((1, TRK, K), lambda b, i: (b, i, 0)),
        out_shape=jax.ShapeDtypeStruct((B, N, K), jnp.int32),
    )(cp, ct)


@functools.lru_cache(maxsize=None)
def _gather_kernel(ntab):
    mesh = plsc.VectorSubcoreMesh(core_axis_name="c", subcore_axis_name="s")

    @functools.partial(
        pl.kernel,
        mesh=mesh,
        out_type=[jax.ShapeDtypeStruct((BNK, C), jnp.float32)] * ntab,
        scratch_types=[pltpu.VMEM((_CH,), jnp.int32)] * 2
        + [pltpu.VMEM((_CH, C), jnp.float32)] * (2 * ntab)
        + [pltpu.SemaphoreType.DMA] * 4,
    )
    def k(*refs):
        tabs = refs[:ntab]
        idx_h = refs[ntab]
        outs = refs[ntab + 1:2 * ntab + 1]
        sc = 2 * ntab + 1
        idx_v = refs[sc:sc + 2]
        rows = [refs[sc + 2 + p * ntab:sc + 2 + (p + 1) * ntab]
                for p in range(2)]
        gsem = refs[sc + 2 + 2 * ntab:sc + 4 + 2 * ntab]
        wsem = refs[sc + 4 + 2 * ntab:sc + 6 + 2 * ntab]
        wid = lax.axis_index("s") * _NC + lax.axis_index("c")
        base = wid * _PER_W

        def off(c):
            return pl.multiple_of(base + c * _CH, _CH)

        def start_gather(c, p):
            pltpu.sync_copy(idx_h.at[pl.ds(off(c), _CH)], idx_v[p])
            for t, r in zip(tabs, rows[p]):
                pltpu.async_copy(t.at[idx_v[p]], r, gsem[p])

        def drain_gather_start_wb(c, p):
            for t, r in zip(tabs, rows[p]):
                pltpu.make_async_copy(t.at[idx_v[p]], r, gsem[p]).wait()
            for r, o in zip(rows[p], outs):
                pltpu.async_copy(r, o.at[pl.ds(off(c), _CH)], wsem[p])

        def drain_wb(c, p):
            for r, o in zip(rows[p], outs):
                pltpu.make_async_copy(r, o.at[pl.ds(off(c), _CH)],
                                      wsem[p]).wait()

        start_gather(0, 0)

        def step(g, carry):
            c0 = 2 * g
            start_gather(c0 + 1, 1)
            drain_gather_start_wb(c0, 0)
            drain_wb(c0, 0)

            @pl.when(g + 1 < _NCH // 2)
            def _():
                start_gather(c0 + 2, 0)
            drain_gather_start_wb(c0 + 1, 1)
            drain_wb(c0 + 1, 1)
            return carry

        lax.fori_loop(0, _NCH // 2, step, 0)

    return k


def _gather1(q, idx_flat):
    res = _gather_kernel(1)(q, idx_flat)
    return res[0] if isinstance(res, (tuple, list)) else res


def _gather2(psi, al, idx_flat):
    return _gather_kernel(2)(psi, al, idx_flat)


def _top0_body(x_ref, w_ref, b_ref, t0_ref, s1_ref, s2_ref):
    i = pl.program_id(0)
    t0 = jnp.dot(x_ref[...], w_ref[...],
                 preferred_element_type=jnp.float32) + b_ref[...]
    t0_ref[...] = t0
    _acc_init(i, s1_ref, s2_ref)
    _acc(s1_ref, s2_ref, t0)


def _top0(x, w, bias):
    return pl.pallas_call(
        _top0_body,
        grid=(_G2,),
        in_specs=[
            pl.BlockSpec((TRL, C), lambda i: (i, 0)),
            pl.BlockSpec((C, C), lambda i: (0, 0)),
            pl.BlockSpec((1, C), lambda i: (0, 0)),
        ],
        out_specs=[
            pl.BlockSpec((TRL, C), lambda i: (i, 0)),
            pl.BlockSpec((1, C), lambda i: (0, 0)),
            pl.BlockSpec((1, C), lambda i: (0, 0)),
        ],
        out_shape=[
            jax.ShapeDtypeStruct((BN_ROWS, C), jnp.float32),
            jax.ShapeDtypeStruct((1, C), jnp.float32),
            jax.ShapeDtypeStruct((1, C), jnp.float32),
        ],
    )(x, w, bias)


def _res_top_body(xp_ref, t5_ref, s5a, s5b, w_ref, b_ref,
                  xn_ref, t0_ref, s1_ref, s2_ref):
    i = pl.program_id(0)
    m, sc = _mstats(s5a, s5b, float(BN_ROWS))
    xn = xp_ref[...] + (t5_ref[...] - m) * sc
    xn_ref[...] = xn
    t0 = jnp.dot(xn, w_ref[...],
                 preferred_element_type=jnp.float32) + b_ref[...]
    t0_ref[...] = t0
    _acc_init(i, s1_ref, s2_ref)
    _acc(s1_ref, s2_ref, t0)


def _res_top(xp, t5, s5a, s5b, w, bias):
    return pl.pallas_call(
        _res_top_body,
        grid=(_G2,),
        in_specs=[
            pl.BlockSpec((TRL, C), lambda i: (i, 0)),
            pl.BlockSpec((TRL, C), lambda i: (i, 0)),
            pl.BlockSpec((1, C), lambda i: (0, 0)),
            pl.BlockSpec((1, C), lambda i: (0, 0)),
            pl.BlockSpec((C, C), lambda i: (0, 0)),
            pl.BlockSpec((1, C), lambda i: (0, 0)),
        ],
        out_specs=[
            pl.BlockSpec((TRL, C), lambda i: (i, 0)),
            pl.BlockSpec((TRL, C), lambda i: (i, 0)),
            pl.BlockSpec((1, C), lambda i: (0, 0)),
            pl.BlockSpec((1, C), lambda i: (0, 0)),
        ],
        out_shape=[
            jax.ShapeDtypeStruct((BN_ROWS, C), jnp.float32),
            jax.ShapeDtypeStruct((BN_ROWS, C), jnp.float32),
            jax.ShapeDtypeStruct((1, C), jnp.float32),
            jax.ShapeDtypeStruct((1, C), jnp.float32),
        ],
    )(xp, t5, s5a, s5b, w, bias)


def _res_final_body(xp_ref, t5_ref, s5a, s5b, out_ref):
    m, sc = _mstats(s5a, s5b, float(BN_ROWS))
    out_ref[...] = xp_ref[...] + (t5_ref[...] - m) * sc


def _res_final(xp, t5, s5a, s5b):
    return pl.pallas_call(
        _res_final_body,
        grid=(_G2,),
        in_specs=[
            pl.BlockSpec((TRL, C), lambda i: (i, 0)),
            pl.BlockSpec((TRL, C), lambda i: (i, 0)),
            pl.BlockSpec((1, C), lambda i: (0, 0)),
            pl.BlockSpec((1, C), lambda i: (0, 0)),
        ],
        out_specs=pl.BlockSpec((TRL, C), lambda i: (i, 0)),
        out_shape=jax.ShapeDtypeStruct((BN_ROWS, C), jnp.float32),
    )(xp, t5, s5a, s5b)


def _qkv_body(t0_ref, s1, s2, cp_ref, wphi, wpsi, wal, wd1,
              phi_ref, psi_ref, al_ref, q_ref):
    m, sc = _mstats(s1, s2, float(BN_ROWS))
    h = (t0_ref[...] - m) * sc
    phi_ref[...] = jnp.dot(h, wphi[...], preferred_element_type=jnp.float32)
    psi_ref[...] = jnp.dot(h, wpsi[...], preferred_element_type=jnp.float32)
    al_ref[...] = jnp.dot(h, wal[...], preferred_element_type=jnp.float32)
    cp = cp_ref[...]
    q_ref[...] = (cp[:, 0:1] * wd1[0:1, :]
                  + cp[:, 1:2] * wd1[1:2, :]
                  + cp[:, 2:3] * wd1[2:3, :])


def _qkv(t0, s1, s2, cpf, wphi, wpsi, wal, wd1p):
    row = pl.BlockSpec((TRL, C), lambda i: (i, 0))
    return pl.pallas_call(
        _qkv_body,
        grid=(_G2,),
        in_specs=[
            row,
            pl.BlockSpec((1, C), lambda i: (0, 0)),
            pl.BlockSpec((1, C), lambda i: (0, 0)),
            pl.BlockSpec((TRL, 8), lambda i: (i, 0)),
            pl.BlockSpec((C, C), lambda i: (0, 0)),
            pl.BlockSpec((C, C), lambda i: (0, 0)),
            pl.BlockSpec((C, C), lambda i: (0, 0)),
            pl.BlockSpec((8, C), lambda i: (0, 0)),
        ],
        out_specs=[row, row, row, row],
        out_shape=[jax.ShapeDtypeStruct((BN_ROWS, C), jnp.float32)] * 4,
    )(t0, s1, s2, cpf, wphi, wpsi, wal, wd1p)


def _t1sum_body(q_ref, qj_ref, s1_ref, s2_ref):
    i = pl.program_id(0)
    t1 = q_ref[...][:, None, :] - qj_ref[...]
    _acc_init(i, s1_ref, s2_ref)
    _acc(s1_ref, s2_ref, t1.reshape(TRB * K, C))


def _t1sum(q, qj):
    return pl.pallas_call(
        _t1sum_body,
        grid=(_G3,),
        in_specs=[
            pl.BlockSpec((TRB, C), lambda i: (i, 0)),
            pl.BlockSpec((TRB, K, C), lambda i: (i, 0, 0)),
        ],
        out_specs=[
            pl.BlockSpec((1, C), lambda i: (0, 0)),
            pl.BlockSpec((1, C), lambda i: (0, 0)),
        ],
        out_shape=[jax.ShapeDtypeStruct((1, C), jnp.float32)] * 2,
    )(q, qj)


def _pe_body(q_ref, qj_ref, s1a, s1b, wd2, t2_ref, s2a_ref, s2b_ref):
    i = pl.program_id(0)
    m, sc = _mstats(s1a, s1b, float(BNK))
    t1 = q_ref[...][:, None, :] - qj_ref[...]
    u = jnp.maximum((t1 - m) * sc, 0.0).reshape(TRB * K, C)
    t2 = jnp.dot(u, wd2[...], preferred_element_type=jnp.float32)
    t2_ref[...] = t2.reshape(TRB, K, C)
    _acc_init(i, s2a_ref, s2b_ref)
    _acc(s2a_ref, s2b_ref, t2)


def _pe(q, qj, s1a, s1b, wd2):
    return pl.pallas_call(
        _pe_body,
        grid=(_G3,),
        in_specs=[
            pl.BlockSpec((TRB, C), lambda i: (i, 0)),
            pl.BlockSpec((TRB, K, C), lambda i: (i, 0, 0)),
            pl.BlockSpec((1, C), lambda i: (0, 0)),
            pl.BlockSpec((1, C), lambda i: (0, 0)),
            pl.BlockSpec((C, C), lambda i: (0, 0)),
        ],
        out_specs=[
            pl.BlockSpec((TRB, K, C), lambda i: (i, 0, 0)),
            pl.BlockSpec((1, C), lambda i: (0, 0)),
            pl.BlockSpec((1, C), lambda i: (0, 0)),
        ],
        out_shape=[
            jax.ShapeDtypeStruct((BN_ROWS, K, C), jnp.float32),
            jax.ShapeDtypeStruct((1, C), jnp.float32),
            jax.ShapeDtypeStruct((1, C), jnp.float32),
        ],
    )(q, qj, s1a, s1b, wd2)


def _g1_body(phi_ref, psij_ref, t2_ref, s2a, s2b, wg1,
             t3_ref, s3a_ref, s3b_ref):
    i = pl.program_id(0)
    m, sc = _mstats(s2a, s2b, float(BNK))
    pe = (t2_ref[...] - m) * sc
    a = phi_ref[...][:, None, :] - psij_ref[...] + pe
    t3 = jnp.dot(a.reshape(TRB * K, C), wg1[...],
                 preferred_element_type=jnp.float32)
    t3_ref[...] = t3.reshape(TRB, K, C)
    _acc_init(i, s3a_ref, s3b_ref)
    _acc(s3a_ref, s3b_ref, t3)


def _g1(phi, psij, t2, s2a, s2b, wg1):
    return pl.pallas_call(
        _g1_body,
        grid=(_G3,),
        in_specs=[
            pl.BlockSpec((TRB, C), lambda i: (i, 0)),
            pl.BlockSpec((TRB, K, C), lambda i: (i, 0, 0)),
            pl.BlockSpec((TRB, K, C), lambda i: (i, 0, 0)),
            pl.BlockSpec((1, C), lambda i: (0, 0)),
            pl.BlockSpec((1, C), lambda i: (0, 0)),
            pl.BlockSpec((C, C), lambda i: (0, 0)),
        ],
        out_specs=[
            pl.BlockSpec((TRB, K, C), lambda i: (i, 0, 0)),
            pl.BlockSpec((1, C), lambda i: (0, 0)),
            pl.BlockSpec((1, C), lambda i: (0, 0)),
        ],
        out_shape=[
            jax.ShapeDtypeStruct((BN_ROWS, K, C), jnp.float32),
            jax.ShapeDtypeStruct((1, C), jnp.float32),
            jax.ShapeDtypeStruct((1, C), jnp.float32),
        ],
    )(phi, psij, t2, s2a, s2b, wg1)


def _g2_body(t3_ref, s3a, s3b, wg2, t4_ref, s4a_ref, s4b_ref):
    i = pl.program_id(0)
    m, sc = _mstats(s3a, s3b, float(BNK))
    u = jnp.maximum((t3_ref[...] - m) * sc, 0.0).reshape(TRB * K, C)
    t4 = jnp.dot(u, wg2[...], preferred_element_type=jnp.float32)
    t4_ref[...] = t4.reshape(TRB, K, C)
    _acc_init(i, s4a_ref, s4b_ref)
    _acc(s4a_ref, s4b_ref, t4)


def _g2(t3, s3a, s3b, wg2):
    return pl.pallas_call(
        _g2_body,
        grid=(_G3,),
        in_specs=[
            pl.BlockSpec((TRB, K, C), lambda i: (i, 0, 0)),
            pl.BlockSpec((1, C), lambda i: (0, 0)),
            pl.BlockSpec((1, C), lambda i: (0, 0)),
            pl.BlockSpec((C, C), lambda i: (0, 0)),
        ],
        out_specs=[
            pl.BlockSpec((TRB, K, C), lambda i: (i, 0, 0)),
            pl.BlockSpec((1, C), lambda i: (0, 0)),
            pl.BlockSpec((1, C), lambda i: (0, 0)),
        ],
        out_shape=[
            jax.ShapeDtypeStruct((BN_ROWS, K, C), jnp.float32),
            jax.ShapeDtypeStruct((1, C), jnp.float32),
            jax.ShapeDtypeStruct((1, C), jnp.float32),
        ],
    )(t3, s3a, s3b, wg2)


def _attn_body(t4_ref, s4a, s4b, t2_ref, s2a, s2b, alj_ref, wdn, bdn,
               t5_ref, s5a_ref, s5b_ref):
    i = pl.program_id(0)
    m4, sc4 = _mstats(s4a, s4b, float(BNK))
    g = (t4_ref[...] - m4) * sc4
    gmax = jnp.max(g, axis=1, keepdims=True)
    e = jnp.exp(g - gmax)
    attn = e / jnp.sum(e, axis=1, keepdims=True)
    m2, sc2 = _mstats(s2a, s2b, float(BNK))
    pe = (t2_ref[...] - m2) * sc2
    y = jnp.sum(attn * (alj_ref[...] + pe), axis=1)
    t5 = jnp.dot(y, wdn[...], preferred_element_type=jnp.float32) + bdn[...]
    t5_ref[...] = t5
    _acc_init(i, s5a_ref, s5b_ref)
    _acc(s5a_ref, s5b_ref, t5)


def _attn(t4, s4a, s4b, t2, s2a, s2b, alj, wdn, bdn):
    return pl.pallas_call(
        _attn_body,
        grid=(_G3,),
        in_specs=[
            pl.BlockSpec((TRB, K, C), lambda i: (i, 0, 0)),
            pl.BlockSpec((1, C), lambda i: (0, 0)),
            pl.BlockSpec((1, C), lambda i: (0, 0)),
            pl.BlockSpec((TRB, K, C), lambda i: (i, 0, 0)),
            pl.BlockSpec((1, C), lambda i: (0, 0)),
            pl.BlockSpec((1, C), lambda i: (0, 0)),
            pl.BlockSpec((TRB, K, C), lambda i: (i, 0, 0)),
            pl.BlockSpec((C, C), lambda i: (0, 0)),
            pl.BlockSpec((1, C), lambda i: (0, 0)),
        ],
        out_specs=[
            pl.BlockSpec((TRB, C), lambda i: (i, 0)),
            pl.BlockSpec((1, C), lambda i: (0, 0)),
            pl.BlockSpec((1, C), lambda i: (0, 0)),
        ],
        out_shape=[
            jax.ShapeDtypeStruct((BN_ROWS, C), jnp.float32),
            jax.ShapeDtypeStruct((1, C), jnp.float32),
            jax.ShapeDtypeStruct((1, C), jnp.float32),
        ],
    )(t4, s4a, s4b, t2, s2a, s2b, alj, wdn, bdn)


def kernel(coords, feats, W_top, b_top, W_phi, W_psi, W_alpha,
           W_g1, W_g2, W_d1, W_d2, W_down, b_down):
    cp = jnp.pad(coords, ((0, 0), (0, 0), (0, 5)))
    ct = jnp.swapaxes(cp, 1, 2)
    idx = _knn(cp, ct)
    idx_flat = idx.reshape(BNK)
    cpf = cp.reshape(BN_ROWS, 8)
    x = feats.reshape(BN_ROWS, C)

    xcur = x
    t5 = s5a = s5b = None
    for blk in range(NBLK):
        wd1p = jnp.pad(W_d1[blk], ((0, 5), (0, 0)))
        bt = b_top[blk].reshape(1, C)
        bd = b_down[blk].reshape(1, C)
        if blk == 0:
            t0, s0a, s0b = _top0(xcur, W_top[blk], bt)
        else:
            xcur, t0, s0a, s0b = _res_top(xcur, t5, s5a, s5b, W_top[blk], bt)
        phi, psi, al, q = _qkv(t0, s0a, s0b, cpf,
                               W_phi[blk], W_psi[blk], W_alpha[blk], wd1p)
        qj = _gather1(q, idx_flat)
        psij, alj = _gather2(psi, al, idx_flat)
        psij = psij.reshape(BN_ROWS, K, C)
        alj = alj.reshape(BN_ROWS, K, C)
        qj = qj.reshape(BN_ROWS, K, C)
        s1a, s1b = _t1sum(q, qj)
        t2, s2a, s2b = _pe(q, qj, s1a, s1b, W_d2[blk])
        t3, s3a, s3b = _g1(phi, psij, t2, s2a, s2b, W_g1[blk])
        t4, s4a, s4b = _g2(t3, s3a, s3b, W_g2[blk])
        t5, s5a, s5b = _attn(t4, s4a, s4b, t2, s2a, s2b, alj,
                             W_down[blk], bd)
    out = _res_final(xcur, t5, s5a, s5b)
    return out.reshape(B, N, C)


# speedup vs baseline: 1.0720x; 1.0720x over previous
import functools

import jax
import jax.numpy as jnp
from jax import lax
from jax.experimental import pallas as pl
from jax.experimental.pallas import tpu as pltpu
from jax.experimental.pallas import tpu_sc as plsc

B, N, C, K, NBLK = 2, 4096, 128, 16, 2
BN_ROWS = B * N
BNK = B * N * K
EPS = 1e-5
TRL = 512
TRB = 512
TRK = 512

_G2 = BN_ROWS // TRL
_G3 = BN_ROWS // TRB

_NC, _NS = 2, 16
_NW = _NC * _NS
_PER_W = BNK // _NW
_CH = 128
_NCH = _PER_W // _CH


def _mstats(s1_ref, s2_ref, n):
    inv = 1.0 / n
    m = s1_ref[...] * inv
    v = s2_ref[...] * inv - m * m
    return m, lax.rsqrt(v + EPS)


def _acc_init(i, s1_ref, s2_ref):
    @pl.when(i == 0)
    def _():
        s1_ref[...] = jnp.zeros_like(s1_ref)
        s2_ref[...] = jnp.zeros_like(s2_ref)


def _acc(s1_ref, s2_ref, x2):
    s1_ref[...] += jnp.sum(x2, axis=0, keepdims=True)
    s2_ref[...] += jnp.sum(x2 * x2, axis=0, keepdims=True)


def _knn_body(cp_ref, ct_ref, idx_ref):
    b = pl.program_id(0)
    cr = cp_ref[0]
    ct = ct_ref[0]
    dot = jnp.dot(cr, ct, preferred_element_type=jnp.float32)
    sq_all = jnp.sum(ct * ct, axis=0, keepdims=True)
    sq_r = jnp.sum(cr * cr, axis=1, keepdims=True)
    d2 = sq_r + sq_all - 2.0 * dot
    jcol = lax.broadcasted_iota(jnp.int32, (TRK, N), 1)
    cols = []
    for _ in range(K):
        m = jnp.min(d2, axis=1, keepdims=True)
        cand = jnp.where(d2 == m, jcol, N)
        ik = jnp.min(cand, axis=1, keepdims=True)
        cols.append(ik)
        d2 = jnp.where(jcol == ik, jnp.inf, d2)
    idx_ref[0] = jnp.concatenate(cols, axis=1) + b * N


def _knn(cp, ct):
    return pl.pallas_call(
        _knn_body,
        grid=(B, N // TRK),
        in_specs=[
            pl.BlockSpec((1, TRK, 8), lambda b, i: (b, i, 0)),
            pl.BlockSpec((1, 8, N), lambda b, i: (b, 0, 0)),
        ],
        out_specs=pl.BlockSpec((1, TRK, K), lambda b, i: (b, i, 0)),
        out_shape=jax.ShapeDtypeStruct((B, N, K), jnp.int32),
    )(cp, ct)


@functools.lru_cache(maxsize=None)
def _gather_kernel(ntab):
    mesh = plsc.VectorSubcoreMesh(core_axis_name="c", subcore_axis_name="s")

    @functools.partial(
        pl.kernel,
        mesh=mesh,
        out_type=[jax.ShapeDtypeStruct((BNK, C), jnp.float32)] * ntab,
        scratch_types=[pltpu.VMEM((_CH,), jnp.int32)] * 2
        + [pltpu.VMEM((_CH, C), jnp.float32)] * (2 * ntab)
        + [pltpu.SemaphoreType.DMA] * 4,
    )
    def k(*refs):
        tabs = refs[:ntab]
        idx_h = refs[ntab]
        outs = refs[ntab + 1:2 * ntab + 1]
        sc = 2 * ntab + 1
        idx_v = refs[sc:sc + 2]
        rows = [refs[sc + 2 + p * ntab:sc + 2 + (p + 1) * ntab]
                for p in range(2)]
        gsem = refs[sc + 2 + 2 * ntab:sc + 4 + 2 * ntab]
        wsem = refs[sc + 4 + 2 * ntab:sc + 6 + 2 * ntab]
        wid = lax.axis_index("s") * _NC + lax.axis_index("c")
        base = wid * _PER_W

        def off(c):
            return pl.multiple_of(base + c * _CH, _CH)

        def start_gather(c, p):
            pltpu.sync_copy(idx_h.at[pl.ds(off(c), _CH)], idx_v[p])
            for t, r in zip(tabs, rows[p]):
                pltpu.async_copy(t.at[idx_v[p]], r, gsem[p])

        def drain_gather_start_wb(c, p):
            for t, r in zip(tabs, rows[p]):
                pltpu.make_async_copy(t.at[idx_v[p]], r, gsem[p]).wait()
            for r, o in zip(rows[p], outs):
                pltpu.async_copy(r, o.at[pl.ds(off(c), _CH)], wsem[p])

        def drain_wb(c, p):
            for r, o in zip(rows[p], outs):
                pltpu.make_async_copy(r, o.at[pl.ds(off(c), _CH)],
                                      wsem[p]).wait()

        start_gather(0, 0)

        def step(g, carry):
            c0 = 2 * g
            start_gather(c0 + 1, 1)
            drain_gather_start_wb(c0, 0)
            drain_wb(c0, 0)

            @pl.when(g + 1 < _NCH // 2)
            def _():
                start_gather(c0 + 2, 0)
            drain_gather_start_wb(c0 + 1, 1)
            drain_wb(c0 + 1, 1)
            return carry

        lax.fori_loop(0, _NCH // 2, step, 0)

    return k


def _gather1(q, idx_flat):
    res = _gather_kernel(1)(q, idx_flat)
    return res[0] if isinstance(res, (tuple, list)) else res


def _gather2(psi, al, idx_flat):
    return _gather_kernel(2)(psi, al, idx_flat)


def _top0_body(x_ref, w_ref, b_ref, t0_ref, s1_ref, s2_ref):
    i = pl.program_id(0)
    t0 = jnp.dot(x_ref[...], w_ref[...],
                 preferred_element_type=jnp.float32) + b_ref[...]
    t0_ref[...] = t0
    _acc_init(i, s1_ref, s2_ref)
    _acc(s1_ref, s2_ref, t0)


def _top0(x, w, bias):
    return pl.pallas_call(
        _top0_body,
        grid=(_G2,),
        in_specs=[
            pl.BlockSpec((TRL, C), lambda i: (i, 0)),
            pl.BlockSpec((C, C), lambda i: (0, 0)),
            pl.BlockSpec((1, C), lambda i: (0, 0)),
        ],
        out_specs=[
            pl.BlockSpec((TRL, C), lambda i: (i, 0)),
            pl.BlockSpec((1, C), lambda i: (0, 0)),
            pl.BlockSpec((1, C), lambda i: (0, 0)),
        ],
        out_shape=[
            jax.ShapeDtypeStruct((BN_ROWS, C), jnp.float32),
            jax.ShapeDtypeStruct((1, C), jnp.float32),
            jax.ShapeDtypeStruct((1, C), jnp.float32),
        ],
    )(x, w, bias)


def _res_top_body(xp_ref, t5_ref, s5a, s5b, w_ref, b_ref,
                  xn_ref, t0_ref, s1_ref, s2_ref):
    i = pl.program_id(0)
    m, sc = _mstats(s5a, s5b, float(BN_ROWS))
    xn = xp_ref[...] + (t5_ref[...] - m) * sc
    xn_ref[...] = xn
    t0 = jnp.dot(xn, w_ref[...],
                 preferred_element_type=jnp.float32) + b_ref[...]
    t0_ref[...] = t0
    _acc_init(i, s1_ref, s2_ref)
    _acc(s1_ref, s2_ref, t0)


def _res_top(xp, t5, s5a, s5b, w, bias):
    return pl.pallas_call(
        _res_top_body,
        grid=(_G2,),
        in_specs=[
            pl.BlockSpec((TRL, C), lambda i: (i, 0)),
            pl.BlockSpec((TRL, C), lambda i: (i, 0)),
            pl.BlockSpec((1, C), lambda i: (0, 0)),
            pl.BlockSpec((1, C), lambda i: (0, 0)),
            pl.BlockSpec((C, C), lambda i: (0, 0)),
            pl.BlockSpec((1, C), lambda i: (0, 0)),
        ],
        out_specs=[
            pl.BlockSpec((TRL, C), lambda i: (i, 0)),
            pl.BlockSpec((TRL, C), lambda i: (i, 0)),
            pl.BlockSpec((1, C), lambda i: (0, 0)),
            pl.BlockSpec((1, C), lambda i: (0, 0)),
        ],
        out_shape=[
            jax.ShapeDtypeStruct((BN_ROWS, C), jnp.float32),
            jax.ShapeDtypeStruct((BN_ROWS, C), jnp.float32),
            jax.ShapeDtypeStruct((1, C), jnp.float32),
            jax.ShapeDtypeStruct((1, C), jnp.float32),
        ],
    )(xp, t5, s5a, s5b, w, bias)


def _res_final_body(xp_ref, t5_ref, s5a, s5b, out_ref):
    m, sc = _mstats(s5a, s5b, float(BN_ROWS))
    out_ref[...] = xp_ref[...] + (t5_ref[...] - m) * sc


def _res_final(xp, t5, s5a, s5b):
    return pl.pallas_call(
        _res_final_body,
        grid=(_G2,),
        in_specs=[
            pl.BlockSpec((TRL, C), lambda i: (i, 0)),
            pl.BlockSpec((TRL, C), lambda i: (i, 0)),
            pl.BlockSpec((1, C), lambda i: (0, 0)),
            pl.BlockSpec((1, C), lambda i: (0, 0)),
        ],
        out_specs=pl.BlockSpec((TRL, C), lambda i: (i, 0)),
        out_shape=jax.ShapeDtypeStruct((BN_ROWS, C), jnp.float32),
    )(xp, t5, s5a, s5b)


def _qkv_body(t0_ref, s1, s2, cp_ref, wphi, wpsi, wal, wd1,
              phi_ref, psi_ref, al_ref, q_ref):
    m, sc = _mstats(s1, s2, float(BN_ROWS))
    h = (t0_ref[...] - m) * sc
    phi_ref[...] = jnp.dot(h, wphi[...], preferred_element_type=jnp.float32)
    psi_ref[...] = jnp.dot(h, wpsi[...], preferred_element_type=jnp.float32)
    al_ref[...] = jnp.dot(h, wal[...], preferred_element_type=jnp.float32)
    cp = cp_ref[...]
    q_ref[...] = (cp[:, 0:1] * wd1[0:1, :]
                  + cp[:, 1:2] * wd1[1:2, :]
                  + cp[:, 2:3] * wd1[2:3, :])


def _qkv(t0, s1, s2, cpf, wphi, wpsi, wal, wd1p):
    row = pl.BlockSpec((TRL, C), lambda i: (i, 0))
    return pl.pallas_call(
        _qkv_body,
        grid=(_G2,),
        in_specs=[
            row,
            pl.BlockSpec((1, C), lambda i: (0, 0)),
            pl.BlockSpec((1, C), lambda i: (0, 0)),
            pl.BlockSpec((TRL, 8), lambda i: (i, 0)),
            pl.BlockSpec((C, C), lambda i: (0, 0)),
            pl.BlockSpec((C, C), lambda i: (0, 0)),
            pl.BlockSpec((C, C), lambda i: (0, 0)),
            pl.BlockSpec((8, C), lambda i: (0, 0)),
        ],
        out_specs=[row, row, row, row],
        out_shape=[jax.ShapeDtypeStruct((BN_ROWS, C), jnp.float32)] * 4,
    )(t0, s1, s2, cpf, wphi, wpsi, wal, wd1p)


def _t1sum_body(q_ref, qj_ref, s1_ref, s2_ref):
    i = pl.program_id(0)
    t1 = q_ref[...][:, None, :] - qj_ref[...]
    _acc_init(i, s1_ref, s2_ref)
    _acc(s1_ref, s2_ref, t1.reshape(TRB * K, C))


def _t1sum(q, qj):
    return pl.pallas_call(
        _t1sum_body,
        grid=(_G3,),
        in_specs=[
            pl.BlockSpec((TRB, C), lambda i: (i, 0)),
            pl.BlockSpec((TRB, K, C), lambda i: (i, 0, 0)),
        ],
        out_specs=[
            pl.BlockSpec((1, C), lambda i: (0, 0)),
            pl.BlockSpec((1, C), lambda i: (0, 0)),
        ],
        out_shape=[jax.ShapeDtypeStruct((1, C), jnp.float32)] * 2,
    )(q, qj)


def _pe_body(q_ref, qj_ref, s1a, s1b, wd2, t2_ref, s2a_ref, s2b_ref):
    i = pl.program_id(0)
    m, sc = _mstats(s1a, s1b, float(BNK))
    t1 = q_ref[...][:, None, :] - qj_ref[...]
    u = jnp.maximum((t1 - m) * sc, 0.0).reshape(TRB * K, C)
    t2 = jnp.dot(u, wd2[...], preferred_element_type=jnp.float32)
    t2_ref[...] = t2.reshape(TRB, K, C)
    _acc_init(i, s2a_ref, s2b_ref)
    _acc(s2a_ref, s2b_ref, t2)


def _pe(q, qj, s1a, s1b, wd2):
    return pl.pallas_call(
        _pe_body,
        grid=(_G3,),
        in_specs=[
            pl.BlockSpec((TRB, C), lambda i: (i, 0)),
            pl.BlockSpec((TRB, K, C), lambda i: (i, 0, 0)),
            pl.BlockSpec((1, C), lambda i: (0, 0)),
            pl.BlockSpec((1, C), lambda i: (0, 0)),
            pl.BlockSpec((C, C), lambda i: (0, 0)),
        ],
        out_specs=[
            pl.BlockSpec((TRB, K, C), lambda i: (i, 0, 0)),
            pl.BlockSpec((1, C), lambda i: (0, 0)),
            pl.BlockSpec((1, C), lambda i: (0, 0)),
        ],
        out_shape=[
            jax.ShapeDtypeStruct((BN_ROWS, K, C), jnp.float32),
            jax.ShapeDtypeStruct((1, C), jnp.float32),
            jax.ShapeDtypeStruct((1, C), jnp.float32),
        ],
    )(q, qj, s1a, s1b, wd2)


def _g1_body(phi_ref, psij_ref, t2_ref, s2a, s2b, wg1,
             t3_ref, s3a_ref, s3b_ref):
    i = pl.program_id(0)
    m, sc = _mstats(s2a, s2b, float(BNK))
    pe = (t2_ref[...] - m) * sc
    a = phi_ref[...][:, None, :] - psij_ref[...] + pe
    t3 = jnp.dot(a.reshape(TRB * K, C), wg1[...],
                 preferred_element_type=jnp.float32)
    t3_ref[...] = t3.reshape(TRB, K, C)
    _acc_init(i, s3a_ref, s3b_ref)
    _acc(s3a_ref, s3b_ref, t3)


def _g1(phi, psij, t2, s2a, s2b, wg1):
    return pl.pallas_call(
        _g1_body,
        grid=(_G3,),
        in_specs=[
            pl.BlockSpec((TRB, C), lambda i: (i, 0)),
            pl.BlockSpec((TRB, K, C), lambda i: (i, 0, 0)),
            pl.BlockSpec((TRB, K, C), lambda i: (i, 0, 0)),
            pl.BlockSpec((1, C), lambda i: (0, 0)),
            pl.BlockSpec((1, C), lambda i: (0, 0)),
            pl.BlockSpec((C, C), lambda i: (0, 0)),
        ],
        out_specs=[
            pl.BlockSpec((TRB, K, C), lambda i: (i, 0, 0)),
            pl.BlockSpec((1, C), lambda i: (0, 0)),
            pl.BlockSpec((1, C), lambda i: (0, 0)),
        ],
        out_shape=[
            jax.ShapeDtypeStruct((BN_ROWS, K, C), jnp.float32),
            jax.ShapeDtypeStruct((1, C), jnp.float32),
            jax.ShapeDtypeStruct((1, C), jnp.float32),
        ],
    )(phi, psij, t2, s2a, s2b, wg1)


def _g2_body(t3_ref, s3a, s3b, wg2, t4_ref, s4a_ref, s4b_ref):
    i = pl.program_id(0)
    m, sc = _mstats(s3a, s3b, float(BNK))
    u = jnp.maximum((t3_ref[...] - m) * sc, 0.0).reshape(TRB * K, C)
    t4 = jnp.dot(u, wg2[...], preferred_element_type=jnp.float32)
    t4_ref[...] = t4.reshape(TRB, K, C)
    _acc_init(i, s4a_ref, s4b_ref)
    _acc(s4a_ref, s4b_ref, t4)


def _g2(t3, s3a, s3b, wg2):
    return pl.pallas_call(
        _g2_body,
        grid=(_G3,),
        in_specs=[
            pl.BlockSpec((TRB, K, C), lambda i: (i, 0, 0)),
            pl.BlockSpec((1, C), lambda i: (0, 0)),
            pl.BlockSpec((1, C), lambda i: (0, 0)),
            pl.BlockSpec((C, C), lambda i: (0, 0)),
        ],
        out_specs=[
            pl.BlockSpec((TRB, K, C), lambda i: (i, 0, 0)),
            pl.BlockSpec((1, C), lambda i: (0, 0)),
            pl.BlockSpec((1, C), lambda i: (0, 0)),
        ],
        out_shape=[
            jax.ShapeDtypeStruct((BN_ROWS, K, C), jnp.float32),
            jax.ShapeDtypeStruct((1, C), jnp.float32),
            jax.ShapeDtypeStruct((1, C), jnp.float32),
        ],
    )(t3, s3a, s3b, wg2)


def _attn_body(t4_ref, s4a, s4b, t2_ref, s2a, s2b, alj_ref, wdn, bdn,
               t5_ref, s5a_ref, s5b_ref):
    i = pl.program_id(0)
    m4, sc4 = _mstats(s4a, s4b, float(BNK))
    g = (t4_ref[...] - m4) * sc4
    gmax = jnp.max(g, axis=1, keepdims=True)
    e = jnp.exp(g - gmax)
    attn = e / jnp.sum(e, axis=1, keepdims=True)
    m2, sc2 = _mstats(s2a, s2b, float(BNK))
    pe = (t2_ref[...] - m2) * sc2
    y = jnp.sum(attn * (alj_ref[...] + pe), axis=1)
    t5 = jnp.dot(y, wdn[...], preferred_element_type=jnp.float32) + bdn[...]
    t5_ref[...] = t5
    _acc_init(i, s5a_ref, s5b_ref)
    _acc(s5a_ref, s5b_ref, t5)


def _attn(t4, s4a, s4b, t2, s2a, s2b, alj, wdn, bdn):
    return pl.pallas_call(
        _attn_body,
        grid=(_G3,),
        in_specs=[
            pl.BlockSpec((TRB, K, C), lambda i: (i, 0, 0)),
            pl.BlockSpec((1, C), lambda i: (0, 0)),
            pl.BlockSpec((1, C), lambda i: (0, 0)),
            pl.BlockSpec((TRB, K, C), lambda i: (i, 0, 0)),
            pl.BlockSpec((1, C), lambda i: (0, 0)),
            pl.BlockSpec((1, C), lambda i: (0, 0)),
            pl.BlockSpec((TRB, K, C), lambda i: (i, 0, 0)),
            pl.BlockSpec((C, C), lambda i: (0, 0)),
            pl.BlockSpec((1, C), lambda i: (0, 0)),
        ],
        out_specs=[
            pl.BlockSpec((TRB, C), lambda i: (i, 0)),
            pl.BlockSpec((1, C), lambda i: (0, 0)),
            pl.BlockSpec((1, C), lambda i: (0, 0)),
        ],
        out_shape=[
            jax.ShapeDtypeStruct((BN_ROWS, C), jnp.float32),
            jax.ShapeDtypeStruct((1, C), jnp.float32),
            jax.ShapeDtypeStruct((1, C), jnp.float32),
        ],
    )(t4, s4a, s4b, t2, s2a, s2b, alj, wdn, bdn)


def kernel(coords, feats, W_top, b_top, W_phi, W_psi, W_alpha,
           W_g1, W_g2, W_d1, W_d2, W_down, b_down):
    cp = jnp.pad(coords, ((0, 0), (0, 0), (0, 5)))
    ct = jnp.swapaxes(cp, 1, 2)
    idx = _knn(cp, ct)
    idx_flat = idx.reshape(BNK)
    cpf = cp.reshape(BN_ROWS, 8)
    x = feats.reshape(BN_ROWS, C)

    xcur = x
    t5 = s5a = s5b = None
    for blk in range(NBLK):
        wd1p = jnp.pad(W_d1[blk], ((0, 5), (0, 0)))
        bt = b_top[blk].reshape(1, C)
        bd = b_down[blk].reshape(1, C)
        if blk == 0:
            t0, s0a, s0b = _top0(xcur, W_top[blk], bt)
        else:
            xcur, t0, s0a, s0b = _res_top(xcur, t5, s5a, s5b, W_top[blk], bt)
        phi, psi, al, q = _qkv(t0, s0a, s0b, cpf,
                               W_phi[blk], W_psi[blk], W_alpha[blk], wd1p)
        qj = _gather1(q, idx_flat)
        psij, alj = _gather2(psi, al, idx_flat)
        psij = psij.reshape(BN_ROWS, K, C)
        alj = alj.reshape(BN_ROWS, K, C)
        qj = qj.reshape(BN_ROWS, K, C)
        s1a, s1b = _t1sum(q, qj)
        t2, s2a, s2b = _pe(q, qj, s1a, s1b, W_d2[blk])
        t3, s3a, s3b = _g1(phi, psij, t2, s2a, s2b, W_g1[blk])
        t4, s4a, s4b = _g2(t3, s3a, s3b, W_g2[blk])
        t5, s5a, s5b = _attn(t4, s4a, s4b, t2, s2a, s2b, alj,
                             W_down[blk], bd)
    out = _res_final(xcur, t5, s5a, s5b)
    return out.reshape(B, N, C)
